# Initial kernel scaffold; baseline (speedup 1.0000x reference)
import jax
import jax.numpy as jnp
from jax.experimental import pallas as pl


def kernel(x, edge_index_gat, edge_type_gat, batch, W_emb, b_emb, W0, q0, k0, bb0, W1, q1, k1, bb1, Wm1, bm1, Wm2, bm2):
    raise NotImplementedError("write your pallas kernel here")


# speedup vs baseline: 34.4894x; 34.4894x over previous
import functools

import jax
import jax.numpy as jnp
from jax import lax
from jax.experimental import pallas as pl
from jax.experimental.pallas import tpu as pltpu
import jax.experimental.pallas.tpu_sc as plsc

N = 10000
E = 320000
R = 4
F_IN = 128
HID = 64
G = 16

NC = 2
NS = 16
NW = NC * NS
EPW = E // NW
CH = 80
NCHUNK = EPW // CH
RPT = N // NS
NQ = HID // 16
BN = 2000
TW = HID


def _idx_body(src_ref, dst_ref, et_ref, a_ref, b_ref, d_ref):
    src = src_ref[...]
    dst = dst_ref[...]
    et = et_ref[...]
    a_ref[...] = et * N + dst
    b_ref[...] = et * N + src
    d_ref[...] = dst


def _edge_indices(src2, dst2, et2):
    return pl.pallas_call(
        _idx_body,
        out_shape=[jax.ShapeDtypeStruct(src2.shape, jnp.int32)] * 3,
    )(src2, dst2, et2)


def _dense_tail(h, w_ref, q_ref, k_ref, xw_ref, qk_ref):
    n = h.shape[0]
    zeros = jnp.zeros((n, 14), jnp.float32)
    for r in range(R):
        xw = jnp.dot(h, w_ref[r], preferred_element_type=jnp.float32)
        xw_ref[r] = xw
        qi = jnp.dot(xw, q_ref[...], preferred_element_type=jnp.float32)
        kj = jnp.dot(xw, k_ref[...], preferred_element_type=jnp.float32)
        qk_ref[r] = jnp.concatenate([qi, kj, zeros], axis=1)


def _pre0_body(x_ref, wemb_ref, bemb_ref, w_ref, q_ref, k_ref,
               xw_ref, qk_ref):
    h = jnp.dot(x_ref[...], wemb_ref[...],
                preferred_element_type=jnp.float32) + bemb_ref[...]
    _dense_tail(h, w_ref, q_ref, k_ref, xw_ref, qk_ref)


def _sred_body(sd_ref, s_ref):
    ones = jnp.ones((NW, 1), jnp.float32)
    s_ref[...] = lax.dot_general(
        sd_ref[...], ones, (((0,), (0,)), ((), ())),
        preferred_element_type=jnp.float32)


def _call_sred(sd):
    return pl.pallas_call(
        _sred_body, out_shape=jax.ShapeDtypeStruct((N, 1), jnp.float32),
    )(sd)


def _combine(o_ref, s, bb_ref):
    num = o_ref[0] + o_ref[1]
    return jnp.maximum(num / (s + 1e-16) + bb_ref[...], 0.0)


def _mid_body(o_ref, sd_ref, bb_ref, w_ref, q_ref, k_ref,
              xw_ref, qk_ref):
    h = _combine(o_ref, sd_ref[...], bb_ref)
    _dense_tail(h, w_ref, q_ref, k_ref, xw_ref, qk_ref)


def _fin_body(o_ref, sd_ref, bb_ref, batch_ref, wm1_ref, bm1_ref,
              wm2_ref, bm2_ref, out_ref):
    h = _combine(o_ref, sd_ref[...], bb_ref)
    b = batch_ref[...]
    pooled = []
    counts = []
    for g in range(G):
        m = b == g
        pooled.append(jnp.sum(jnp.where(m, h, 0.0), axis=0, keepdims=True))
        counts.append(jnp.sum(m.astype(jnp.float32), axis=0, keepdims=True))
    p = jnp.concatenate(pooled, axis=0)
    c = jnp.concatenate(counts, axis=0)
    p = p / jnp.maximum(c, 1.0)
    hm = jnp.maximum(
        jnp.dot(p, wm1_ref[...], preferred_element_type=jnp.float32)
        + bm1_ref[...], 0.0)
    out_ref[...] = jnp.dot(hm, wm2_ref[...],
                           preferred_element_type=jnp.float32) + bm2_ref[...]


def _sc_body(xw_hbm, qk_hbm, a_hbm, b_hbm, d_hbm, out_hbm, sd_hbm,
             rows_v, qka_v, qkb_v, e_v, a_v, b_v, d_v, s_v, acc,
             sem, sem2):
    cid = lax.axis_index("c")
    sid = lax.axis_index("s")
    wid = cid * NS + sid

    def _zero(rr, carry):
        for q in range(NQ):
            rows_v[rr, pl.ds(16 * q, 16)] = jnp.zeros((16,), jnp.float32)
        return carry
    lax.fori_loop(0, CH, _zero, 0)

    def _zs(ii, carry):
        s_v[pl.ds(16 * ii, 16)] = jnp.zeros((16,), jnp.float32)
        return carry
    lax.fori_loop(0, N // 16, _zs, 0)
    nrc = N // CH
    for t in range((nrc + NS - 1) // NS):
        c = sid + NS * t
        off = pl.multiple_of(c * CH, 8)

        @pl.when(c < nrc)
        def _():
            pltpu.sync_copy(rows_v, acc.at[pl.ds(off, CH)])
    plsc.subcore_barrier()

    ebase = wid * EPW
    lanes = lax.iota(jnp.int32, 16)
    col0 = jnp.zeros((16,), jnp.int32)
    col1 = col0 + 1

    def _chunk(i, carry):
        off = pl.multiple_of(ebase + i * CH, 8)
        pltpu.sync_copy(a_hbm.at[pl.ds(off, CH)], a_v)
        pltpu.sync_copy(b_hbm.at[pl.ds(off, CH)], b_v)
        pltpu.sync_copy(d_hbm.at[pl.ds(off, CH)], d_v)
        cpr = pltpu.async_copy(xw_hbm.at[b_v], rows_v, sem)
        cpa = pltpu.async_copy(qk_hbm.at[a_v], qka_v, sem2)
        cpb = pltpu.async_copy(qk_hbm.at[b_v], qkb_v, sem2)
        cpa.wait()
        cpb.wait()
        for j in range(CH // 16):
            rl = lanes + 16 * j
            qi16 = plsc.load_gather(qka_v, [rl, col0])
            kj16 = plsc.load_gather(qkb_v, [rl, col1])
            ee = jnp.exp(qi16 * kj16)
            e_v[pl.ds(16 * j, 16)] = ee
            dv = d_v[pl.ds(16 * j, 16)]
            plsc.addupdate_scatter(s_v, [dv], ee)
        cpr.wait()

        def _scale(jj, c2):
            ew = e_v[pl.ds(16 * jj, 16)]
            for ri in range(16):
                rr = 16 * jj + ri
                w = ew[ri]
                for q in range(NQ):
                    rows_v[rr, pl.ds(16 * q, 16)] = (
                        rows_v[rr, pl.ds(16 * q, 16)] * w)
            return c2
        lax.fori_loop(0, CH // 16, _scale, 0)
        pltpu.sync_copy(rows_v, acc.at[d_v], add=True)
        return carry
    lax.fori_loop(0, NCHUNK, _chunk, 0)
    plsc.subcore_barrier()

    for t in range((nrc + NS - 1) // NS):
        c = sid + NS * t
        off = pl.multiple_of(c * CH, 8)
        oof = pl.multiple_of(cid * N + c * CH, 8)

        @pl.when(c < nrc)
        def _():
            pltpu.sync_copy(acc.at[pl.ds(off, CH)],
                            out_hbm.at[pl.ds(oof, CH)])
    pltpu.sync_copy(s_v, sd_hbm.at[wid])


@functools.partial(
    pl.kernel,
    out_type=(jax.ShapeDtypeStruct((NC * N, TW), jnp.float32),
              jax.ShapeDtypeStruct((NW, N), jnp.float32)),
    mesh=plsc.VectorSubcoreMesh(core_axis_name="c", subcore_axis_name="s",
                                num_cores=NC, num_subcores=NS),
    scratch_types=[
        pltpu.VMEM((CH, TW), jnp.float32),
        pltpu.VMEM((CH, 16), jnp.float32),
        pltpu.VMEM((CH, 16), jnp.float32),
        pltpu.VMEM((CH,), jnp.float32),
        pltpu.VMEM((CH,), jnp.int32),
        pltpu.VMEM((CH,), jnp.int32),
        pltpu.VMEM((CH,), jnp.int32),
        pltpu.VMEM((N,), jnp.float32),
        pltpu.VMEM_SHARED((N, TW), jnp.float32),
        pltpu.SemaphoreType.DMA,
        pltpu.SemaphoreType.DMA,
    ],
    compiler_params=pltpu.CompilerParams(needs_layout_passes=False,
                                         use_tc_tiling_on_sc=False),
)
def _sc_layer(xw_hbm, qk_hbm, a_hbm, b_hbm, d_hbm, out_hbm, sd_hbm,
              rows_v, qka_v, qkb_v, e_v, a_v, b_v, d_v, s_v, acc, sem, sem2):
    _sc_body(xw_hbm, qk_hbm, a_hbm, b_hbm, d_hbm, out_hbm, sd_hbm,
             rows_v, qka_v, qkb_v, e_v, a_v, b_v, d_v, s_v, acc, sem, sem2)


_XW_SHAPE = jax.ShapeDtypeStruct((R, N, TW), jnp.float32)
_QK_SHAPE = jax.ShapeDtypeStruct((R, N, 16), jnp.float32)
_W_SPECS = [
    pl.BlockSpec((1, HID), lambda i: (0, 0)),
    pl.BlockSpec((R, HID, HID), lambda i: (0, 0, 0)),
    pl.BlockSpec((HID, 1), lambda i: (0, 0)),
    pl.BlockSpec((HID, 1), lambda i: (0, 0)),
]
_OUT_SPECS = [
    pl.BlockSpec((R, BN, TW), lambda i: (0, i, 0)),
    pl.BlockSpec((R, BN, 16), lambda i: (0, i, 0)),
]


def _call_pre0(x, W_emb, b_emb, W0, q0, k0):
    return pl.pallas_call(
        _pre0_body, out_shape=[_XW_SHAPE, _QK_SHAPE],
        grid=(N // BN,),
        in_specs=[
            pl.BlockSpec((BN, F_IN), lambda i: (i, 0)),
            pl.BlockSpec((F_IN, HID), lambda i: (0, 0)),
        ] + _W_SPECS,
        out_specs=_OUT_SPECS,
    )(x, W_emb, b_emb.reshape(1, HID), W0, q0, k0)


def _call_mid(o0, sd0, bb0, W1, q1, k1):
    s0 = _call_sred(sd0)
    return pl.pallas_call(
        _mid_body, out_shape=[_XW_SHAPE, _QK_SHAPE],
        grid=(N // BN,),
        in_specs=[
            pl.BlockSpec((NC, BN, TW), lambda i: (0, i, 0)),
            pl.BlockSpec((BN, 1), lambda i: (i, 0)),
        ] + _W_SPECS,
        out_specs=_OUT_SPECS,
    )(o0.reshape(NC, N, TW), s0, bb0.reshape(1, HID), W1, q1, k1)


def _call_fin(o1, sd1, bb1, batch, Wm1, bm1, Wm2, bm2):
    s1 = _call_sred(sd1)
    return pl.pallas_call(
        _fin_body, out_shape=jax.ShapeDtypeStruct((G, 1), jnp.float32),
    )(o1.reshape(NC, N, TW), s1, bb1.reshape(1, HID),
      batch.reshape(N, 1).astype(jnp.int32),
      Wm1, bm1.reshape(1, HID), Wm2, bm2.reshape(1, 1))


def kernel(x, edge_index_gat, edge_type_gat, batch, W_emb, b_emb,
           W0, q0, k0, bb0, W1, q1, k1, bb1, Wm1, bm1, Wm2, bm2):
    src2 = edge_index_gat[0].reshape(E // 128, 128).astype(jnp.int32)
    dst2 = edge_index_gat[1].reshape(E // 128, 128).astype(jnp.int32)
    et2 = edge_type_gat.reshape(E // 128, 128).astype(jnp.int32)
    a2, b2, d2 = _edge_indices(src2, dst2, et2)
    A = a2.reshape(E)
    B = b2.reshape(E)
    D = d2.reshape(E)

    xw0, qk0 = _call_pre0(x, W_emb, b_emb, W0, q0, k0)
    o0, sd0 = _sc_layer(xw0.reshape(R * N, TW), qk0.reshape(R * N, 16),
                        A, B, D)
    xw1, qk1 = _call_mid(o0, sd0, bb0, W1, q1, k1)
    o1, sd1 = _sc_layer(xw1.reshape(R * N, TW), qk1.reshape(R * N, 16),
                        A, B, D)
    out = _call_fin(o1, sd1, bb1, batch, Wm1, bm1, Wm2, bm2)
    return out.reshape(G)


# speedup vs baseline: 43.6645x; 1.2660x over previous
import functools

import jax
import jax.numpy as jnp
from jax import lax
from jax.experimental import pallas as pl
from jax.experimental.pallas import tpu as pltpu
import jax.experimental.pallas.tpu_sc as plsc

N = 10000
E = 320000
R = 4
F_IN = 128
HID = 64
G = 16

NC = 2
NS = 16
NW = NC * NS
EPW = E // NW
CH = 80
NCHUNK = EPW // CH
RPT = N // NS
NQ = HID // 16
BN = 2000
TW = HID


def _idx_body(src_ref, dst_ref, et_ref, a_ref, b_ref, d_ref):
    src = src_ref[...]
    dst = dst_ref[...]
    et = et_ref[...]
    a_ref[...] = et * N + dst
    b_ref[...] = et * N + src
    d_ref[...] = dst


def _edge_indices(src2, dst2, et2):
    return pl.pallas_call(
        _idx_body,
        out_shape=[jax.ShapeDtypeStruct(src2.shape, jnp.int32)] * 3,
    )(src2, dst2, et2)


def _dense_tail(h, w_ref, q_ref, k_ref, xw_ref, qk_ref):
    n = h.shape[0]
    zeros = jnp.zeros((n, 14), jnp.float32)
    for r in range(R):
        xw = jnp.dot(h, w_ref[r], preferred_element_type=jnp.float32)
        xw_ref[r] = xw
        qi = jnp.dot(xw, q_ref[...], preferred_element_type=jnp.float32)
        kj = jnp.dot(xw, k_ref[...], preferred_element_type=jnp.float32)
        qk_ref[r] = jnp.concatenate([qi, kj, zeros], axis=1)


def _pre0_body(x_ref, wemb_ref, bemb_ref, w_ref, q_ref, k_ref,
               xw_ref, qk_ref):
    h = jnp.dot(x_ref[...], wemb_ref[...],
                preferred_element_type=jnp.float32) + bemb_ref[...]
    _dense_tail(h, w_ref, q_ref, k_ref, xw_ref, qk_ref)


def _sred_body(sd_ref, s_ref):
    ones = jnp.ones((NW, 1), jnp.float32)
    s_ref[...] = lax.dot_general(
        sd_ref[...], ones, (((0,), (0,)), ((), ())),
        preferred_element_type=jnp.float32)


def _call_sred(sd):
    return pl.pallas_call(
        _sred_body, out_shape=jax.ShapeDtypeStruct((N, 1), jnp.float32),
    )(sd)


def _combine(o_ref, s, bb_ref):
    num = o_ref[0] + o_ref[1]
    return jnp.maximum(num / (s + 1e-16) + bb_ref[...], 0.0)


def _mid_body(o_ref, sd_ref, bb_ref, w_ref, q_ref, k_ref,
              xw_ref, qk_ref):
    h = _combine(o_ref, sd_ref[...], bb_ref)
    _dense_tail(h, w_ref, q_ref, k_ref, xw_ref, qk_ref)


def _fin_body(o_ref, sd_ref, bb_ref, batch_ref, wm1_ref, bm1_ref,
              wm2_ref, bm2_ref, out_ref):
    h = _combine(o_ref, sd_ref[...], bb_ref)
    b = batch_ref[...]
    pooled = []
    counts = []
    for g in range(G):
        m = b == g
        pooled.append(jnp.sum(jnp.where(m, h, 0.0), axis=0, keepdims=True))
        counts.append(jnp.sum(m.astype(jnp.float32), axis=0, keepdims=True))
    p = jnp.concatenate(pooled, axis=0)
    c = jnp.concatenate(counts, axis=0)
    p = p / jnp.maximum(c, 1.0)
    hm = jnp.maximum(
        jnp.dot(p, wm1_ref[...], preferred_element_type=jnp.float32)
        + bm1_ref[...], 0.0)
    out_ref[...] = jnp.dot(hm, wm2_ref[...],
                           preferred_element_type=jnp.float32) + bm2_ref[...]


def _sc_body(xw_hbm, qk_hbm, a_hbm, b_hbm, d_hbm, out_hbm, sd_hbm,
             rows0, rows1, qka0, qka1, qkb0, qkb1, e_v,
             a0, a1, b0, b1, d0, d1, s_v, acc,
             semr0, semr1, semq0, semq1, sems0, sems1):
    cid = lax.axis_index("c")
    sid = lax.axis_index("s")
    wid = cid * NS + sid
    slots = ((rows0, qka0, qkb0, a0, b0, d0, semr0, semq0, sems0),
             (rows1, qka1, qkb1, a1, b1, d1, semr1, semq1, sems1))

    def _zero(rr, carry):
        for q in range(NQ):
            rows0[rr, pl.ds(16 * q, 16)] = jnp.zeros((16,), jnp.float32)
        return carry
    lax.fori_loop(0, CH, _zero, 0)

    def _zs(ii, carry):
        s_v[pl.ds(16 * ii, 16)] = jnp.zeros((16,), jnp.float32)
        return carry
    lax.fori_loop(0, N // 16, _zs, 0)
    nrc = N // CH
    for t in range((nrc + NS - 1) // NS):
        c = sid + NS * t
        off = pl.multiple_of(c * CH, 8)

        @pl.when(c < nrc)
        def _():
            pltpu.sync_copy(rows0, acc.at[pl.ds(off, CH)])
    plsc.subcore_barrier()

    ebase = wid * EPW
    lanes = lax.iota(jnp.int32, 16)
    col0 = jnp.zeros((16,), jnp.int32)
    col1 = col0 + 1

    def _prime(c, slot):
        rows_v, qka_v, qkb_v, a_v, b_v, d_v, semr, semq, sems = slot
        off = pl.multiple_of(ebase + c * CH, 8)
        pltpu.sync_copy(a_hbm.at[pl.ds(off, CH)], a_v)
        pltpu.sync_copy(b_hbm.at[pl.ds(off, CH)], b_v)
        pltpu.sync_copy(d_hbm.at[pl.ds(off, CH)], d_v)
        pltpu.async_copy(xw_hbm.at[b_v], rows_v, semr)
        pltpu.async_copy(qk_hbm.at[a_v], qka_v, semq)
        pltpu.async_copy(qk_hbm.at[b_v], qkb_v, semq)

    def _wait_scatter(slot):
        rows_v, _, _, _, _, d_v, _, _, sems = slot
        pltpu.make_async_copy(rows_v, acc.at[d_v], sems).wait()

    def _process(slot):
        rows_v, qka_v, qkb_v, a_v, b_v, d_v, semr, semq, sems = slot
        pltpu.make_async_copy(qk_hbm.at[a_v], qka_v, semq).wait()
        pltpu.make_async_copy(qk_hbm.at[b_v], qkb_v, semq).wait()
        for j in range(CH // 16):
            rl = lanes + 16 * j
            qi16 = plsc.load_gather(qka_v, [rl, col0])
            kj16 = plsc.load_gather(qkb_v, [rl, col1])
            ee = jnp.exp(qi16 * kj16)
            e_v[pl.ds(16 * j, 16)] = ee
            dv = d_v[pl.ds(16 * j, 16)]
            plsc.addupdate_scatter(s_v, [dv], ee)
        pltpu.make_async_copy(xw_hbm.at[b_v], rows_v, semr).wait()

        def _scale(jj, c2):
            ew = e_v[pl.ds(16 * jj, 16)]
            for ri in range(16):
                rr = 16 * jj + ri
                w = ew[ri]
                for q in range(NQ):
                    rows_v[rr, pl.ds(16 * q, 16)] = (
                        rows_v[rr, pl.ds(16 * q, 16)] * w)
            return c2
        lax.fori_loop(0, CH // 16, _scale, 0)
        pltpu.async_copy(rows_v, acc.at[d_v], sems, add=True)

    _prime(0, slots[0])
    _prime(1, slots[1])

    def _pair(g, carry):
        c0 = 2 * g
        _process(slots[0])

        @pl.when(c0 + 2 < NCHUNK)
        def _():
            _wait_scatter(slots[0])
            _prime(c0 + 2, slots[0])
        _process(slots[1])

        @pl.when(c0 + 3 < NCHUNK)
        def _():
            _wait_scatter(slots[1])
            _prime(c0 + 3, slots[1])
        return carry
    lax.fori_loop(0, NCHUNK // 2, _pair, 0)
    if NCHUNK % 2:
        _process(slots[0])
    _wait_scatter(slots[0])
    _wait_scatter(slots[1])
    plsc.subcore_barrier()

    for t in range((nrc + NS - 1) // NS):
        c = sid + NS * t
        off = pl.multiple_of(c * CH, 8)
        oof = pl.multiple_of(cid * N + c * CH, 8)

        @pl.when(c < nrc)
        def _():
            pltpu.sync_copy(acc.at[pl.ds(off, CH)],
                            out_hbm.at[pl.ds(oof, CH)])
    pltpu.sync_copy(s_v, sd_hbm.at[wid])


@functools.partial(
    pl.kernel,
    out_type=(jax.ShapeDtypeStruct((NC * N, TW), jnp.float32),
              jax.ShapeDtypeStruct((NW, N), jnp.float32)),
    mesh=plsc.VectorSubcoreMesh(core_axis_name="c", subcore_axis_name="s",
                                num_cores=NC, num_subcores=NS),
    scratch_types=[
        pltpu.VMEM((CH, TW), jnp.float32),
        pltpu.VMEM((CH, TW), jnp.float32),
        pltpu.VMEM((CH, 16), jnp.float32),
        pltpu.VMEM((CH, 16), jnp.float32),
        pltpu.VMEM((CH, 16), jnp.float32),
        pltpu.VMEM((CH, 16), jnp.float32),
        pltpu.VMEM((CH,), jnp.float32),
        pltpu.VMEM((CH,), jnp.int32),
        pltpu.VMEM((CH,), jnp.int32),
        pltpu.VMEM((CH,), jnp.int32),
        pltpu.VMEM((CH,), jnp.int32),
        pltpu.VMEM((CH,), jnp.int32),
        pltpu.VMEM((CH,), jnp.int32),
        pltpu.VMEM((N,), jnp.float32),
        pltpu.VMEM_SHARED((N, TW), jnp.float32),
        pltpu.SemaphoreType.DMA,
        pltpu.SemaphoreType.DMA,
        pltpu.SemaphoreType.DMA,
        pltpu.SemaphoreType.DMA,
        pltpu.SemaphoreType.DMA,
        pltpu.SemaphoreType.DMA,
    ],
    compiler_params=pltpu.CompilerParams(needs_layout_passes=False,
                                         use_tc_tiling_on_sc=False),
)
def _sc_layer(xw_hbm, qk_hbm, a_hbm, b_hbm, d_hbm, out_hbm, sd_hbm,
              rows0, rows1, qka0, qka1, qkb0, qkb1, e_v,
              a0, a1, b0, b1, d0, d1, s_v, acc,
              semr0, semr1, semq0, semq1, sems0, sems1):
    _sc_body(xw_hbm, qk_hbm, a_hbm, b_hbm, d_hbm, out_hbm, sd_hbm,
             rows0, rows1, qka0, qka1, qkb0, qkb1, e_v,
             a0, a1, b0, b1, d0, d1, s_v, acc,
             semr0, semr1, semq0, semq1, sems0, sems1)


_XW_SHAPE = jax.ShapeDtypeStruct((R, N, TW), jnp.float32)
_QK_SHAPE = jax.ShapeDtypeStruct((R, N, 16), jnp.float32)
_W_SPECS = [
    pl.BlockSpec((1, HID), lambda i: (0, 0)),
    pl.BlockSpec((R, HID, HID), lambda i: (0, 0, 0)),
    pl.BlockSpec((HID, 1), lambda i: (0, 0)),
    pl.BlockSpec((HID, 1), lambda i: (0, 0)),
]
_OUT_SPECS = [
    pl.BlockSpec((R, BN, TW), lambda i: (0, i, 0)),
    pl.BlockSpec((R, BN, 16), lambda i: (0, i, 0)),
]


def _call_pre0(x, W_emb, b_emb, W0, q0, k0):
    return pl.pallas_call(
        _pre0_body, out_shape=[_XW_SHAPE, _QK_SHAPE],
        grid=(N // BN,),
        in_specs=[
            pl.BlockSpec((BN, F_IN), lambda i: (i, 0)),
            pl.BlockSpec((F_IN, HID), lambda i: (0, 0)),
        ] + _W_SPECS,
        out_specs=_OUT_SPECS,
    )(x, W_emb, b_emb.reshape(1, HID), W0, q0, k0)


def _call_mid(o0, sd0, bb0, W1, q1, k1):
    s0 = _call_sred(sd0)
    return pl.pallas_call(
        _mid_body, out_shape=[_XW_SHAPE, _QK_SHAPE],
        grid=(N // BN,),
        in_specs=[
            pl.BlockSpec((NC, BN, TW), lambda i: (0, i, 0)),
            pl.BlockSpec((BN, 1), lambda i: (i, 0)),
        ] + _W_SPECS,
        out_specs=_OUT_SPECS,
    )(o0.reshape(NC, N, TW), s0, bb0.reshape(1, HID), W1, q1, k1)


def _call_fin(o1, sd1, bb1, batch, Wm1, bm1, Wm2, bm2):
    s1 = _call_sred(sd1)
    return pl.pallas_call(
        _fin_body, out_shape=jax.ShapeDtypeStruct((G, 1), jnp.float32),
    )(o1.reshape(NC, N, TW), s1, bb1.reshape(1, HID),
      batch.reshape(N, 1).astype(jnp.int32),
      Wm1, bm1.reshape(1, HID), Wm2, bm2.reshape(1, 1))


def kernel(x, edge_index_gat, edge_type_gat, batch, W_emb, b_emb,
           W0, q0, k0, bb0, W1, q1, k1, bb1, Wm1, bm1, Wm2, bm2):
    src2 = edge_index_gat[0].reshape(E // 128, 128).astype(jnp.int32)
    dst2 = edge_index_gat[1].reshape(E // 128, 128).astype(jnp.int32)
    et2 = edge_type_gat.reshape(E // 128, 128).astype(jnp.int32)
    a2, b2, d2 = _edge_indices(src2, dst2, et2)
    A = a2.reshape(E)
    B = b2.reshape(E)
    D = d2.reshape(E)

    xw0, qk0 = _call_pre0(x, W_emb, b_emb, W0, q0, k0)
    o0, sd0 = _sc_layer(xw0.reshape(R * N, TW), qk0.reshape(R * N, 16),
                        A, B, D)
    xw1, qk1 = _call_mid(o0, sd0, bb0, W1, q1, k1)
    o1, sd1 = _sc_layer(xw1.reshape(R * N, TW), qk1.reshape(R * N, 16),
                        A, B, D)
    out = _call_fin(o1, sd1, bb1, batch, Wm1, bm1, Wm2, bm2)
    return out.reshape(G)


# speedup vs baseline: 57.8499x; 1.3249x over previous
import functools

import jax
import jax.numpy as jnp
from jax import lax
from jax.experimental import pallas as pl
from jax.experimental.pallas import tpu as pltpu
import jax.experimental.pallas.tpu_sc as plsc

N = 10000
E = 320000
R = 4
F_IN = 128
HID = 64
G = 16

NC = 2
NS = 16
NW = NC * NS
EPW = E // NW
CH = 80
NCHUNK = EPW // CH
RPT = N // NS
NQ = HID // 16
BN = 2000
TW = HID


def _idx_body(src_ref, dst_ref, et_ref, abd_ref):
    src = src_ref[...]
    dst = dst_ref[...]
    et = et_ref[...]
    abd_ref[:, 0, :] = et * N + dst
    abd_ref[:, 1, :] = et * N + src
    abd_ref[:, 2, :] = dst


def _edge_indices(src2, dst2, et2):
    return pl.pallas_call(
        _idx_body,
        out_shape=jax.ShapeDtypeStruct((E // CH, 3, CH), jnp.int32),
    )(src2, dst2, et2)


def _dense_tail(h, w_ref, q_ref, k_ref, xw_ref, qk_ref):
    n = h.shape[0]
    zeros = jnp.zeros((n, 14), jnp.float32)
    for r in range(R):
        xw = jnp.dot(h, w_ref[r], preferred_element_type=jnp.float32)
        xw_ref[r] = xw
        qi = jnp.dot(xw, q_ref[...], preferred_element_type=jnp.float32)
        kj = jnp.dot(xw, k_ref[...], preferred_element_type=jnp.float32)
        qk_ref[r] = jnp.concatenate([qi, kj, zeros], axis=1)


def _pre0_body(x_ref, wemb_ref, bemb_ref, w_ref, q_ref, k_ref,
               xw_ref, qk_ref):
    h = jnp.dot(x_ref[...], wemb_ref[...],
                preferred_element_type=jnp.float32) + bemb_ref[...]
    _dense_tail(h, w_ref, q_ref, k_ref, xw_ref, qk_ref)


def _sred_body(sd_ref, s_ref):
    ones = jnp.ones((NW, 1), jnp.float32)
    s_ref[...] = lax.dot_general(
        sd_ref[...], ones, (((0,), (0,)), ((), ())),
        preferred_element_type=jnp.float32)


def _call_sred(sd):
    return pl.pallas_call(
        _sred_body, out_shape=jax.ShapeDtypeStruct((N, 1), jnp.float32),
    )(sd)


def _combine(o_ref, s, bb_ref):
    num = o_ref[0] + o_ref[1]
    return jnp.maximum(num / (s + 1e-16) + bb_ref[...], 0.0)


def _mid_body(o_ref, sd_ref, bb_ref, w_ref, q_ref, k_ref,
              xw_ref, qk_ref):
    h = _combine(o_ref, sd_ref[...], bb_ref)
    _dense_tail(h, w_ref, q_ref, k_ref, xw_ref, qk_ref)


def _fin_body(o_ref, sd_ref, bb_ref, batch_ref, wm1_ref, bm1_ref,
              wm2_ref, bm2_ref, out_ref):
    h = _combine(o_ref, sd_ref[...], bb_ref)
    b = batch_ref[...]
    pooled = []
    counts = []
    for g in range(G):
        m = b == g
        pooled.append(jnp.sum(jnp.where(m, h, 0.0), axis=0, keepdims=True))
        counts.append(jnp.sum(m.astype(jnp.float32), axis=0, keepdims=True))
    p = jnp.concatenate(pooled, axis=0)
    c = jnp.concatenate(counts, axis=0)
    p = p / jnp.maximum(c, 1.0)
    hm = jnp.maximum(
        jnp.dot(p, wm1_ref[...], preferred_element_type=jnp.float32)
        + bm1_ref[...], 0.0)
    out_ref[...] = jnp.dot(hm, wm2_ref[...],
                           preferred_element_type=jnp.float32) + bm2_ref[...]


def _sc_body(xw_hbm, qk_hbm, abd_hbm, out_hbm, sd_hbm,
             rows0, rows1, qka0, qka1, qkb0, qkb1, e_v,
             i0, i1, s_v, acc,
             semr0, semr1, semq0, semq1, sems0, sems1):
    cid = lax.axis_index("c")
    sid = lax.axis_index("s")
    wid = cid * NS + sid
    slots = ((rows0, qka0, qkb0, i0, semr0, semq0, sems0),
             (rows1, qka1, qkb1, i1, semr1, semq1, sems1))

    def _zero(rr, carry):
        for q in range(NQ):
            rows0[rr, pl.ds(16 * q, 16)] = jnp.zeros((16,), jnp.float32)
        return carry
    lax.fori_loop(0, CH, _zero, 0)

    def _zs(ii, carry):
        s_v[pl.ds(16 * ii, 16)] = jnp.zeros((16,), jnp.float32)
        return carry
    lax.fori_loop(0, N // 16, _zs, 0)
    nrc = N // CH
    for t in range((nrc + NS - 1) // NS):
        c = sid + NS * t
        off = pl.multiple_of(c * CH, 8)

        @pl.when(c < nrc)
        def _():
            pltpu.sync_copy(rows0, acc.at[pl.ds(off, CH)])
    plsc.subcore_barrier()

    ebase = wid * EPW
    lanes = lax.iota(jnp.int32, 16)
    col0 = jnp.zeros((16,), jnp.int32)
    col1 = col0 + 1

    def _prime(c, slot):
        rows_v, qka_v, qkb_v, i_v, semr, semq, sems = slot
        chk = wid * NCHUNK + c
        pltpu.sync_copy(abd_hbm.at[chk], i_v)
        pltpu.async_copy(xw_hbm.at[i_v.at[1]], rows_v, semr)
        pltpu.async_copy(qk_hbm.at[i_v.at[0]], qka_v, semq)
        pltpu.async_copy(qk_hbm.at[i_v.at[1]], qkb_v, semq)

    def _wait_scatter(slot):
        rows_v, _, _, i_v, _, _, sems = slot
        pltpu.make_async_copy(rows_v, acc.at[i_v.at[2]], sems).wait()

    def _process(slot):
        rows_v, qka_v, qkb_v, i_v, semr, semq, sems = slot
        pltpu.make_async_copy(qk_hbm.at[i_v.at[0]], qka_v, semq).wait()
        pltpu.make_async_copy(qk_hbm.at[i_v.at[1]], qkb_v, semq).wait()
        for j in range(CH // 16):
            rl = lanes + 16 * j
            qi16 = plsc.load_gather(qka_v, [rl, col0])
            kj16 = plsc.load_gather(qkb_v, [rl, col1])
            ee = jnp.exp(qi16 * kj16)
            e_v[pl.ds(16 * j, 16)] = ee
            dv = i_v[2, pl.ds(16 * j, 16)]
            plsc.addupdate_scatter(s_v, [dv], ee)
        pltpu.make_async_copy(xw_hbm.at[i_v.at[1]], rows_v, semr).wait()

        def _scale(jj, c2):
            ew = e_v[pl.ds(16 * jj, 16)]
            for ri in range(16):
                rr = 16 * jj + ri
                w = ew[ri]
                for q in range(NQ):
                    rows_v[rr, pl.ds(16 * q, 16)] = (
                        rows_v[rr, pl.ds(16 * q, 16)] * w)
            return c2
        lax.fori_loop(0, CH // 16, _scale, 0)
        pltpu.async_copy(rows_v, acc.at[i_v.at[2]], sems, add=True)

    _prime(0, slots[0])
    _prime(1, slots[1])

    def _pair(g, carry):
        c0 = 2 * g
        _process(slots[0])

        @pl.when(c0 + 2 < NCHUNK)
        def _():
            _wait_scatter(slots[0])
            _prime(c0 + 2, slots[0])
        _process(slots[1])

        @pl.when(c0 + 3 < NCHUNK)
        def _():
            _wait_scatter(slots[1])
            _prime(c0 + 3, slots[1])
        return carry
    lax.fori_loop(0, NCHUNK // 2, _pair, 0)
    if NCHUNK % 2:
        _process(slots[0])
    _wait_scatter(slots[0])
    _wait_scatter(slots[1])
    plsc.subcore_barrier()

    for t in range((nrc + NS - 1) // NS):
        c = sid + NS * t
        off = pl.multiple_of(c * CH, 8)
        oof = pl.multiple_of(cid * N + c * CH, 8)

        @pl.when(c < nrc)
        def _():
            pltpu.sync_copy(acc.at[pl.ds(off, CH)],
                            out_hbm.at[pl.ds(oof, CH)])
    pltpu.sync_copy(s_v, sd_hbm.at[wid])


@functools.partial(
    pl.kernel,
    out_type=(jax.ShapeDtypeStruct((NC * N, TW), jnp.float32),
              jax.ShapeDtypeStruct((NW, N), jnp.float32)),
    mesh=plsc.VectorSubcoreMesh(core_axis_name="c", subcore_axis_name="s",
                                num_cores=NC, num_subcores=NS),
    scratch_types=[
        pltpu.VMEM((CH, TW), jnp.float32),
        pltpu.VMEM((CH, TW), jnp.float32),
        pltpu.VMEM((CH, 16), jnp.float32),
        pltpu.VMEM((CH, 16), jnp.float32),
        pltpu.VMEM((CH, 16), jnp.float32),
        pltpu.VMEM((CH, 16), jnp.float32),
        pltpu.VMEM((CH,), jnp.float32),
        pltpu.VMEM((3, CH), jnp.int32),
        pltpu.VMEM((3, CH), jnp.int32),
        pltpu.VMEM((N,), jnp.float32),
        pltpu.VMEM_SHARED((N, TW), jnp.float32),
        pltpu.SemaphoreType.DMA,
        pltpu.SemaphoreType.DMA,
        pltpu.SemaphoreType.DMA,
        pltpu.SemaphoreType.DMA,
        pltpu.SemaphoreType.DMA,
        pltpu.SemaphoreType.DMA,
    ],
    compiler_params=pltpu.CompilerParams(needs_layout_passes=False,
                                         use_tc_tiling_on_sc=False),
)
def _sc_layer(xw_hbm, qk_hbm, abd_hbm, out_hbm, sd_hbm,
              rows0, rows1, qka0, qka1, qkb0, qkb1, e_v,
              i0, i1, s_v, acc,
              semr0, semr1, semq0, semq1, sems0, sems1):
    _sc_body(xw_hbm, qk_hbm, abd_hbm, out_hbm, sd_hbm,
             rows0, rows1, qka0, qka1, qkb0, qkb1, e_v,
             i0, i1, s_v, acc,
             semr0, semr1, semq0, semq1, sems0, sems1)


_XW_SHAPE = jax.ShapeDtypeStruct((R, N, TW), jnp.float32)
_QK_SHAPE = jax.ShapeDtypeStruct((R, N, 16), jnp.float32)
_W_SPECS = [
    pl.BlockSpec((1, HID), lambda i: (0, 0)),
    pl.BlockSpec((R, HID, HID), lambda i: (0, 0, 0)),
    pl.BlockSpec((HID, 1), lambda i: (0, 0)),
    pl.BlockSpec((HID, 1), lambda i: (0, 0)),
]
_OUT_SPECS = [
    pl.BlockSpec((R, BN, TW), lambda i: (0, i, 0)),
    pl.BlockSpec((R, BN, 16), lambda i: (0, i, 0)),
]


def _call_pre0(x, W_emb, b_emb, W0, q0, k0):
    return pl.pallas_call(
        _pre0_body, out_shape=[_XW_SHAPE, _QK_SHAPE],
        grid=(N // BN,),
        in_specs=[
            pl.BlockSpec((BN, F_IN), lambda i: (i, 0)),
            pl.BlockSpec((F_IN, HID), lambda i: (0, 0)),
        ] + _W_SPECS,
        out_specs=_OUT_SPECS,
    )(x, W_emb, b_emb.reshape(1, HID), W0, q0, k0)


def _call_mid(o0, sd0, bb0, W1, q1, k1):
    s0 = _call_sred(sd0)
    return pl.pallas_call(
        _mid_body, out_shape=[_XW_SHAPE, _QK_SHAPE],
        grid=(N // BN,),
        in_specs=[
            pl.BlockSpec((NC, BN, TW), lambda i: (0, i, 0)),
            pl.BlockSpec((BN, 1), lambda i: (i, 0)),
        ] + _W_SPECS,
        out_specs=_OUT_SPECS,
    )(o0.reshape(NC, N, TW), s0, bb0.reshape(1, HID), W1, q1, k1)


def _call_fin(o1, sd1, bb1, batch, Wm1, bm1, Wm2, bm2):
    s1 = _call_sred(sd1)
    return pl.pallas_call(
        _fin_body, out_shape=jax.ShapeDtypeStruct((G, 1), jnp.float32),
    )(o1.reshape(NC, N, TW), s1, bb1.reshape(1, HID),
      batch.reshape(N, 1).astype(jnp.int32),
      Wm1, bm1.reshape(1, HID), Wm2, bm2.reshape(1, 1))


def kernel(x, edge_index_gat, edge_type_gat, batch, W_emb, b_emb,
           W0, q0, k0, bb0, W1, q1, k1, bb1, Wm1, bm1, Wm2, bm2):
    src2 = edge_index_gat[0].reshape(E // CH, CH).astype(jnp.int32)
    dst2 = edge_index_gat[1].reshape(E // CH, CH).astype(jnp.int32)
    et2 = edge_type_gat.reshape(E // CH, CH).astype(jnp.int32)
    ABD = _edge_indices(src2, dst2, et2)

    xw0, qk0 = _call_pre0(x, W_emb, b_emb, W0, q0, k0)
    o0, sd0 = _sc_layer(xw0.reshape(R * N, TW), qk0.reshape(R * N, 16), ABD)
    xw1, qk1 = _call_mid(o0, sd0, bb0, W1, q1, k1)
    o1, sd1 = _sc_layer(xw1.reshape(R * N, TW), qk1.reshape(R * N, 16), ABD)
    out = _call_fin(o1, sd1, bb1, batch, Wm1, bm1, Wm2, bm2)
    return out.reshape(G)
